# baseline (speedup 1.0000x reference)
import jax
import jax.numpy as jnp
from jax.experimental import pallas as pl
from jax.experimental.pallas import tpu as pltpu

_TOP_K = 8
_NUM_EXPERTS = 64
_EPS = 1e-20
_TEMP = 1.0
_TILE_K = 32768


def _gating_kernel(x_ref, w_ref, bn_ref, o_ref, acc_ref):
    k = pl.program_id(0)
    nk = pl.num_programs(0)

    @pl.when(k == 0)
    def _init():
        acc_ref[...] = jnp.zeros_like(acc_ref)

    acc_ref[...] += jax.lax.dot_general(
        x_ref[...], w_ref[...],
        dimension_numbers=(((1,), (1,)), ((), ())),
        preferred_element_type=jnp.float32)

    @pl.when(k == nk - 1)
    def _epilogue():
        p = acc_ref[...] + bn_ref[...]
        ps = p / _TEMP
        m = jnp.max(ps, axis=-1, keepdims=True)
        e = jnp.exp(ps - m)
        soft = e / jnp.sum(e, axis=-1, keepdims=True)
        t = jnp.max(p, axis=-1, keepdims=True)
        for _ in range(_TOP_K - 1):
            cnt = jnp.sum((p >= t).astype(jnp.int32), axis=-1, keepdims=True)
            nxt = jnp.max(jnp.where(p < t, p, -jnp.inf), axis=-1, keepdims=True)
            t = jnp.where(cnt >= _TOP_K, t, nxt)
        mask = jax.nn.sigmoid((p - t) / _TEMP)
        s = soft * mask
        o_ref[...] = s / jnp.sum(s, axis=-1, keepdims=True)


def kernel(x, W, b):
    B = x.shape[0]
    xf = x.reshape(B, -1)
    K = xf.shape[1]
    nk = K // _TILE_K
    U = jax.random.uniform(jax.random.key(1), (B, _NUM_EXPERTS),
                           dtype=jnp.float32)
    noise = -jnp.log(-jnp.log(U + _EPS) + _EPS)
    bn = b[None, :] + noise

    return pl.pallas_call(
        _gating_kernel,
        grid=(nk,),
        in_specs=[
            pl.BlockSpec((B, _TILE_K), lambda k: (0, k)),
            pl.BlockSpec((_NUM_EXPERTS, _TILE_K), lambda k: (0, k)),
            pl.BlockSpec((B, _NUM_EXPERTS), lambda k: (0, 0)),
        ],
        out_specs=pl.BlockSpec((B, _NUM_EXPERTS), lambda k: (0, 0)),
        out_shape=jax.ShapeDtypeStruct((B, _NUM_EXPERTS), jnp.float32),
        scratch_shapes=[pltpu.VMEM((B, _NUM_EXPERTS), jnp.float32)],
        compiler_params=pltpu.CompilerParams(
            dimension_semantics=("arbitrary",)),
    )(xf, W, bn)
